# baseline (speedup 1.0000x reference)
import functools

import jax
import jax.numpy as jnp
from jax import lax
from jax.experimental import pallas as pl
from jax.experimental.pallas import tpu as pltpu
from jax.experimental.pallas import tpu_sc as plsc

BATCH = 16384
EMB = 32
NSLICE = 4
DSL = EMB // NSLICE

_info = plsc.get_sparse_core_info()
NC, NS, L = _info.num_cores, _info.num_subcores, _info.num_lanes
NW = NC * NS
B_PER_W = BATCH // NW
N_GROUP = B_PER_W // L
N_ELEM = B_PER_W * EMB
SL_ELEM = B_PER_W * DSL
CHUNK = 128
FIRE = 8

_mesh = plsc.VectorSubcoreMesh(core_axis_name="c", subcore_axis_name="s")


@functools.partial(
    pl.kernel,
    mesh=_mesh,
    out_type=jax.ShapeDtypeStruct((BATCH,), jnp.float32),
    scratch_types=[
        pltpu.VMEM((B_PER_W,), jnp.int32),
        pltpu.VMEM((B_PER_W,), jnp.int32),
        pltpu.VMEM((SL_ELEM,), jnp.int32),
        pltpu.VMEM((SL_ELEM,), jnp.int32),
        pltpu.VMEM((N_ELEM,), jnp.float32),
        pltpu.VMEM((N_ELEM,), jnp.float32),
        pltpu.VMEM((B_PER_W,), jnp.float32),
        pltpu.SemaphoreType.DMA,
    ],
    compiler_params=pltpu.CompilerParams(
        needs_layout_passes=False, use_tc_tiling_on_sc=False),
)
def _mf_sc(u_hbm, v_hbm, us0, us1, us2, us3, it0, it1, it2, it3, out_hbm,
           iu, iv, offu, offv, ubuf, vbuf, out_v, sem):
    wid = lax.axis_index("s") * NC + lax.axis_index("c")
    base = wid * B_PER_W

    pltpu.sync_copy(u_hbm.at[pl.ds(base, B_PER_W)], iu)
    pltpu.sync_copy(v_hbm.at[pl.ds(base, B_PER_W)], iv)

    def expand(g, carry):
        iu_vec = iu[pl.ds(g * L, L)]
        iv_vec = iv[pl.ds(g * L, L)]
        for d in range(DSL):
            s = d * B_PER_W + g * L
            offu[pl.ds(s, L)] = iu_vec + d * 1000000
            offv[pl.ds(s, L)] = iv_vec + d * 1000000
        return carry

    lax.fori_loop(0, N_GROUP, expand, 0)

    for k, (us, it) in enumerate([(us0, it0), (us1, it1),
                                  (us2, it2), (us3, it3)]):
        for c0 in range(0, SL_ELEM // CHUNK, FIRE):
            copies = []
            for c in range(c0, c0 + FIRE):
                s = c * CHUNK
                d0 = k * SL_ELEM + s
                copies.append(pltpu.async_copy(
                    us.at[offu.at[pl.ds(s, CHUNK)]],
                    ubuf.at[pl.ds(d0, CHUNK)], sem))
                copies.append(pltpu.async_copy(
                    it.at[offv.at[pl.ds(s, CHUNK)]],
                    vbuf.at[pl.ds(d0, CHUNK)], sem))
            for cp in copies:
                cp.wait()

    def dot(g, carry):
        acc = jnp.zeros((L,), jnp.float32)
        for d in range(EMB):
            s = d * B_PER_W + g * L
            acc = acc + ubuf[pl.ds(s, L)] * vbuf[pl.ds(s, L)]
        out_v[pl.ds(g * L, L)] = acc
        return carry

    lax.fori_loop(0, N_GROUP, dot, 0)

    pltpu.sync_copy(out_v, out_hbm.at[pl.ds(base, B_PER_W)])


def kernel(u, v, users_emb, items_emb):
    ut = users_emb.T
    it = items_emb.T
    us = [ut[DSL * k:DSL * (k + 1)].reshape(-1) for k in range(NSLICE)]
    its = [it[DSL * k:DSL * (k + 1)].reshape(-1) for k in range(NSLICE)]
    return _mf_sc(u.astype(jnp.int32), v.astype(jnp.int32), *us, *its)


# speedup vs baseline: 5.8167x; 5.8167x over previous
import functools

import jax
import jax.numpy as jnp
from jax import lax
from jax.experimental import pallas as pl
from jax.experimental.pallas import tpu as pltpu
from jax.experimental.pallas import tpu_sc as plsc

BATCH = 16384
EMB = 32

_info = plsc.get_sparse_core_info()
NC, NS, L = _info.num_cores, _info.num_subcores, _info.num_lanes
NW = NC * NS
B_PER_W = BATCH // NW
N_CHUNK = B_PER_W // 128
N_GROUP = B_PER_W // L

_mesh = plsc.VectorSubcoreMesh(core_axis_name="c", subcore_axis_name="s")


@functools.partial(
    pl.kernel,
    mesh=_mesh,
    out_type=jax.ShapeDtypeStruct((BATCH,), jnp.float32),
    scratch_types=[
        pltpu.VMEM((N_CHUNK, 128), jnp.int32),
        pltpu.VMEM((N_CHUNK, 128), jnp.int32),
        pltpu.VMEM((B_PER_W, EMB), jnp.float32),
        pltpu.VMEM((B_PER_W, EMB), jnp.float32),
        pltpu.VMEM((B_PER_W,), jnp.float32),
        pltpu.SemaphoreType.DMA,
    ],
    compiler_params=pltpu.CompilerParams(
        needs_layout_passes=False, use_tc_tiling_on_sc=False),
)
def _mf_sc(u_hbm, v_hbm, ue_hbm, ie_hbm, out_hbm,
           idx_u, idx_v, rows_u, rows_v, out_v, sem):
    wid = lax.axis_index("s") * NC + lax.axis_index("c")

    pltpu.sync_copy(u_hbm.at[pl.ds(wid * N_CHUNK, N_CHUNK)], idx_u)
    pltpu.sync_copy(v_hbm.at[pl.ds(wid * N_CHUNK, N_CHUNK)], idx_v)

    copies = []
    for j in range(N_CHUNK):
        copies.append(pltpu.async_copy(
            ue_hbm.at[idx_u.at[j]], rows_u.at[pl.ds(j * 128, 128)], sem))
        copies.append(pltpu.async_copy(
            ie_hbm.at[idx_v.at[j]], rows_v.at[pl.ds(j * 128, 128)], sem))
    for cp in copies:
        cp.wait()

    iota = lax.broadcasted_iota(jnp.int32, (L,), 0)

    def g_body(g, carry):
        row = g * L + iota
        acc = jnp.zeros((L,), jnp.float32)
        for c in range(EMB):
            colv = jnp.full((L,), c, jnp.int32)
            gu = plsc.load_gather(rows_u, [row, colv])
            gv = plsc.load_gather(rows_v, [row, colv])
            acc = acc + gu * gv
        out_v[pl.ds(g * L, L)] = acc
        return carry

    lax.fori_loop(0, N_GROUP, g_body, 0)

    pltpu.sync_copy(out_v, out_hbm.at[pl.ds(wid * B_PER_W, B_PER_W)])


def kernel(u, v, users_emb, items_emb):
    u2 = u.astype(jnp.int32).reshape(BATCH // 128, 128)
    v2 = v.astype(jnp.int32).reshape(BATCH // 128, 128)
    return _mf_sc(u2, v2, users_emb, items_emb)
